# baseline (speedup 1.0000x reference)
import functools

import jax
import jax.numpy as jnp
from jax import lax
from jax.experimental import pallas as pl
from jax.experimental.pallas import tpu as pltpu
from jax.experimental.pallas import tpu_sc as plsc

N = 10000
E = 320000
D = 128
ALPHA = 0.5

K = 128
NT = 16
CPT = 162
NCHP = NT * CPT
EPAD = NCHP * K - E
ACCROWS = 10016
NROWCH = N // 128
NTAIL = N - NROWCH * 128
ZTAIL = ACCROWS - NROWCH * 128

_mesh = plsc.VectorSubcoreMesh(core_axis_name="c", subcore_axis_name="s")


def _zero_acc(zeros_hbm, acc_sp, sid):
    for k in range(5):
        c = sid + k * NT

        @pl.when(c < NROWCH)
        def _():
            pltpu.sync_copy(zeros_hbm, acc_sp.at[pl.ds(c * 128, 128)])

    @pl.when(sid == 1)
    def _():
        pltpu.sync_copy(zeros_hbm.at[pl.ds(0, ZTAIL)],
                        acc_sp.at[pl.ds(NROWCH * 128, ZTAIL)])


def _write_out(acc_sp, out_hbm, sid):
    for k in range(5):
        c = sid + k * NT

        @pl.when(c < NROWCH)
        def _():
            pltpu.sync_copy(acc_sp.at[pl.ds(c * 128, 128)],
                            out_hbm.at[pl.ds(c * 128, 128)])

    @pl.when(sid == 0)
    def _():
        pltpu.sync_copy(acc_sp.at[pl.ds(NROWCH * 128, NTAIL)],
                        out_hbm.at[pl.ds(NROWCH * 128, NTAIL)])


@functools.partial(
    pl.kernel,
    out_type=jax.ShapeDtypeStruct((2, N, D), jnp.float32),
    mesh=_mesh,
    scratch_types=[
        pltpu.VMEM((CPT, K), jnp.int32),
        pltpu.VMEM((K, D), jnp.float32),
        pltpu.VMEM_SHARED((ACCROWS, D), jnp.float32),
        pltpu.SemaphoreType.DMA,
    ],
)
def _deg_kernel(es_hbm, ones_hbm, zeros_hbm, deg_out, idx2, ones_v, acc_sp,
                sem):
    cid = lax.axis_index("c")
    sid = lax.axis_index("s")
    pltpu.sync_copy(ones_hbm, ones_v)
    pltpu.sync_copy(es_hbm.at[1 - cid, sid], idx2)
    _zero_acc(zeros_hbm, acc_sp, sid)
    plsc.subcore_barrier()

    def body(j, carry):
        c = 6 * j
        for b in range(6):
            pltpu.async_copy(ones_v, acc_sp.at[idx2.at[c + b]], sem, add=True)
        for b in range(6):
            pltpu.make_async_copy(ones_v, acc_sp.at[idx2.at[c + b]], sem).wait()
        return carry

    lax.fori_loop(0, CPT // 6, body, 0)
    plsc.subcore_barrier()
    _write_out(acc_sp, deg_out.at[cid], sid)


@functools.partial(
    pl.kernel,
    out_type=(jax.ShapeDtypeStruct((N, D), jnp.float32),
              jax.ShapeDtypeStruct((N, D), jnp.float32)),
    mesh=_mesh,
    scratch_types=[
        pltpu.VMEM((6, 2, K), jnp.int32),
        pltpu.VMEM((3, K, D), jnp.float32),
        pltpu.VMEM_SHARED((ACCROWS, D), jnp.float32),
        pltpu.SemaphoreType.DMA,
        pltpu.SemaphoreType.DMA,
        pltpu.SemaphoreType.DMA,
        pltpu.SemaphoreType.DMA,
        pltpu.SemaphoreType.DMA,
        pltpu.SemaphoreType.DMA,
        pltpu.SemaphoreType.DMA,
        pltpu.SemaphoreType.DMA,
    ],
)
def _conv_kernel(mf_hbm, mb_hbm, eic_hbm, zeros_hbm, accf_out, accb_out,
                 ibuf, gbuf, acc_sp, gsem0, gsem1, gsem2, ssem0, ssem1, ssem2,
                 isem0, isem1):
    cid = lax.axis_index("c")
    sid = lax.axis_index("s")
    gsem = (gsem0, gsem1, gsem2)
    ssem = (ssem0, ssem1, ssem2)
    isem = (isem0, isem1)

    def run(m_hbm, out_hbm):
        _zero_acc(zeros_hbm, acc_sp, sid)
        plsc.subcore_barrier()

        pltpu.sync_copy(eic_hbm.at[cid, sid, 0], ibuf.at[0])
        pltpu.sync_copy(eic_hbm.at[cid, sid, 1], ibuf.at[1])
        pltpu.async_copy(eic_hbm.at[cid, sid, 2], ibuf.at[2], isem[0])
        pltpu.async_copy(eic_hbm.at[cid, sid, 3], ibuf.at[3], isem[1])
        pltpu.async_copy(m_hbm.at[ibuf.at[0, 0]], gbuf.at[0], gsem[0])
        pltpu.async_copy(m_hbm.at[ibuf.at[1, 0]], gbuf.at[1], gsem[1])

        def step(j, q):
            c = 6 * j + q
            b = q % 3
            p2 = q % 2
            pltpu.make_async_copy(m_hbm.at[ibuf.at[q, 0]], gbuf.at[b],
                                  gsem[b]).wait()

            def wait_prev_scatter():
                pltpu.make_async_copy(gbuf.at[(q + 2) % 3],
                                      acc_sp.at[ibuf.at[(q + 5) % 6, 1]],
                                      ssem[(q + 2) % 3]).wait()

            if q == 0:
                @pl.when(j > 0)
                def _():
                    wait_prev_scatter()
            else:
                wait_prev_scatter()

            @pl.when(c + 2 < CPT)
            def _():
                pltpu.make_async_copy(eic_hbm.at[cid, sid, 0],
                                      ibuf.at[(q + 2) % 6], isem[p2]).wait()
                pltpu.async_copy(m_hbm.at[ibuf.at[(q + 2) % 6, 0]],
                                 gbuf.at[(q + 2) % 3], gsem[(q + 2) % 3])

            pltpu.async_copy(gbuf.at[b], acc_sp.at[ibuf.at[q, 1]], ssem[b],
                             add=True)

            @pl.when(c + 4 < CPT)
            def _():
                pltpu.async_copy(eic_hbm.at[cid, sid, c + 4],
                                 ibuf.at[(q + 4) % 6], isem[p2])

        def body(j, carry):
            for q in range(6):
                step(j, q)
            return carry

        lax.fori_loop(0, CPT // 6, body, 0)
        pltpu.make_async_copy(gbuf.at[2], acc_sp.at[ibuf.at[5, 1]],
                              ssem[2]).wait()
        plsc.subcore_barrier()
        _write_out(acc_sp, out_hbm, sid)

    @pl.when(cid == 0)
    def _():
        run(mf_hbm, accf_out)

    @pl.when(cid == 1)
    def _():
        run(mb_hbm, accb_out)


def _scales(deg2):
    ds = lax.rsqrt(jnp.maximum(deg2[:, 0:1], 1.0))
    dd = lax.rsqrt(jnp.maximum(deg2[:, 1:2], 1.0))
    return ds, dd


def _mm0_body(x_ref, deg_ref, wf_ref, wb_ref, mf_ref, mb_ref):
    x = x_ref[...]
    ds, dd = _scales(deg_ref[...])
    mf_ref[...] = jnp.dot(x * ds, wf_ref[...], preferred_element_type=jnp.float32)
    mb_ref[...] = jnp.dot(x * dd, wb_ref[...], preferred_element_type=jnp.float32)


def _mid_body(accf_ref, accb_ref, deg_ref, bf_ref, bb_ref, wf_ref, wb_ref,
              mf_ref, mb_ref):
    ds, dd = _scales(deg_ref[...])
    h = (ALPHA * (accf_ref[...] * dd + bf_ref[...])
         + (1.0 - ALPHA) * (accb_ref[...] * ds + bb_ref[...]))
    h = jnp.maximum(h, 0.0)
    mf_ref[...] = jnp.dot(h * ds, wf_ref[...], preferred_element_type=jnp.float32)
    mb_ref[...] = jnp.dot(h * dd, wb_ref[...], preferred_element_type=jnp.float32)


def _fin_body(accf_ref, accb_ref, deg_ref, bf_ref, bb_ref, out_ref):
    ds, dd = _scales(deg_ref[...])
    out_ref[...] = (ALPHA * (accf_ref[...] * dd + bf_ref[...])
                    + (1.0 - ALPHA) * (accb_ref[...] * ds + bb_ref[...]))


_f32 = jnp.float32
_nd = jax.ShapeDtypeStruct((N, D), _f32)


def kernel(x, edge_index, Wf0, bf0, Wb0, bb0, Wf1, bf1, Wb1, bb1):
    onesD = jnp.ones((K, D), _f32)
    zerosD = jnp.zeros((128, D), _f32)

    src = edge_index[0]
    dst = edge_index[1]
    pad0 = jnp.zeros((EPAD,), jnp.int32)
    padN = jnp.full((EPAD,), N, jnp.int32)
    g0 = jnp.concatenate([src, pad0]).reshape(NCHP, K)
    s0 = jnp.concatenate([dst, padN]).reshape(NCHP, K)
    g1 = jnp.concatenate([dst, pad0]).reshape(NCHP, K)
    s1 = jnp.concatenate([src, padN]).reshape(NCHP, K)
    eic = jnp.stack([jnp.stack([g0, s0], axis=1),
                     jnp.stack([g1, s1], axis=1)])
    eic = eic.reshape(2, NT, CPT, 2, K)
    es = jnp.stack([s0, s1]).reshape(2, NT, CPT, K)

    deg3 = _deg_kernel(es, onesD, zerosD)
    deg2 = jnp.stack([deg3[0, :, 0], deg3[1, :, 0]], axis=1)

    mf0, mb0 = pl.pallas_call(_mm0_body, out_shape=(_nd, _nd))(
        x, deg2, Wf0, Wb0)
    accf0, accb0 = _conv_kernel(mf0, mb0, eic, zerosD)

    mf1, mb1 = pl.pallas_call(_mid_body, out_shape=(_nd, _nd))(
        accf0, accb0, deg2, bf0.reshape(1, D), bb0.reshape(1, D), Wf1, Wb1)
    accf1, accb1 = _conv_kernel(mf1, mb1, eic, zerosD)

    out = pl.pallas_call(_fin_body, out_shape=_nd)(
        accf1, accb1, deg2, bf1.reshape(1, D), bb1.reshape(1, D))
    return out


# speedup vs baseline: 1.8317x; 1.8317x over previous
import functools

import jax
import jax.numpy as jnp
from jax import lax
from jax.experimental import pallas as pl
from jax.experimental.pallas import tpu as pltpu
from jax.experimental.pallas import tpu_sc as plsc

N = 10000
E = 320000
D = 128
ALPHA = 0.5

K = 120
NT = 16
CPT = 168
NB = CPT // 3
NCHP = NT * CPT
EPAD = NCHP * K - E
ACCROWS = 10016
NROWCH = N // 128
NTAIL = N - NROWCH * 128
ZTAIL = ACCROWS - NROWCH * 128

_mesh = plsc.VectorSubcoreMesh(core_axis_name="c", subcore_axis_name="s")


def _zero_acc(zeros_hbm, acc_sp, sid):
    for k in range(5):
        c = sid + k * NT

        @pl.when(c < NROWCH)
        def _():
            pltpu.sync_copy(zeros_hbm, acc_sp.at[pl.ds(c * 128, 128)])

    @pl.when(sid == 1)
    def _():
        pltpu.sync_copy(zeros_hbm.at[pl.ds(0, ZTAIL)],
                        acc_sp.at[pl.ds(NROWCH * 128, ZTAIL)])


def _write_out(acc_sp, out_hbm, sid):
    for k in range(5):
        c = sid + k * NT

        @pl.when(c < NROWCH)
        def _():
            pltpu.sync_copy(acc_sp.at[pl.ds(c * 128, 128)],
                            out_hbm.at[pl.ds(c * 128, 128)])

    @pl.when(sid == 0)
    def _():
        pltpu.sync_copy(acc_sp.at[pl.ds(NROWCH * 128, NTAIL)],
                        out_hbm.at[pl.ds(NROWCH * 128, NTAIL)])


@functools.partial(
    pl.kernel,
    out_type=jax.ShapeDtypeStruct((2, N, D), jnp.float32),
    mesh=_mesh,
    scratch_types=[
        pltpu.VMEM((CPT, K), jnp.int32),
        pltpu.VMEM((K, D), jnp.float32),
        pltpu.VMEM_SHARED((ACCROWS, D), jnp.float32),
        pltpu.SemaphoreType.DMA,
    ],
)
def _deg_kernel(es_hbm, ones_hbm, zeros_hbm, deg_out, idx2, ones_v, acc_sp,
                sem):
    cid = lax.axis_index("c")
    sid = lax.axis_index("s")
    pltpu.sync_copy(ones_hbm, ones_v)
    pltpu.sync_copy(es_hbm.at[1 - cid, sid], idx2)
    _zero_acc(zeros_hbm, acc_sp, sid)
    plsc.subcore_barrier()

    def body(j, carry):
        c = 6 * j
        for b in range(6):
            pltpu.async_copy(ones_v, acc_sp.at[idx2.at[c + b]], sem, add=True)
        for b in range(6):
            pltpu.make_async_copy(ones_v, acc_sp.at[idx2.at[c + b]], sem).wait()
        return carry

    lax.fori_loop(0, CPT // 6, body, 0)
    plsc.subcore_barrier()
    _write_out(acc_sp, deg_out.at[cid], sid)


@functools.partial(
    pl.kernel,
    out_type=(jax.ShapeDtypeStruct((N, D), jnp.float32),
              jax.ShapeDtypeStruct((N, D), jnp.float32)),
    mesh=_mesh,
    scratch_types=[
        pltpu.VMEM((2, 6, K), jnp.int32),
        pltpu.VMEM((3, K, D), jnp.float32),
        pltpu.VMEM_SHARED((ACCROWS, D), jnp.float32),
        pltpu.SemaphoreType.DMA,
        pltpu.SemaphoreType.DMA,
        pltpu.SemaphoreType.DMA,
    ],
)
def _conv_kernel(mf_hbm, mb_hbm, eic_hbm, zeros_hbm, accf_out, accb_out,
                 ibuf, gbuf, acc_sp, gsem, ssem, isem):
    cid = lax.axis_index("c")
    sid = lax.axis_index("s")

    def run(m_hbm, out_hbm):
        _zero_acc(zeros_hbm, acc_sp, sid)
        plsc.subcore_barrier()

        pltpu.sync_copy(eic_hbm.at[cid, sid, 0], ibuf.at[0])

        def batch(a, nb_idx):
            g = [pltpu.async_copy(m_hbm.at[ibuf.at[a, 2 * b]], gbuf.at[b],
                                  gsem) for b in range(3)]
            ipf = pltpu.async_copy(eic_hbm.at[cid, sid, nb_idx],
                                   ibuf.at[1 - a], isem)
            for d in g:
                d.wait()
            s = [pltpu.async_copy(gbuf.at[b], acc_sp.at[ibuf.at[a, 2 * b + 1]],
                                  ssem, add=True) for b in range(3)]
            for d in s:
                d.wait()
            ipf.wait()

        def body(j, carry):
            b0 = 2 * j
            batch(0, b0 + 1)
            batch(1, jnp.minimum(b0 + 2, NB - 1))
            return carry

        lax.fori_loop(0, NB // 2, body, 0)
        plsc.subcore_barrier()
        _write_out(acc_sp, out_hbm, sid)

    @pl.when(cid == 0)
    def _():
        run(mf_hbm, accf_out)

    @pl.when(cid == 1)
    def _():
        run(mb_hbm, accb_out)


def _scales(deg2):
    ds = lax.rsqrt(jnp.maximum(deg2[:, 0:1], 1.0))
    dd = lax.rsqrt(jnp.maximum(deg2[:, 1:2], 1.0))
    return ds, dd


def _mm0_body(x_ref, deg_ref, wf_ref, wb_ref, mf_ref, mb_ref):
    x = x_ref[...]
    ds, dd = _scales(deg_ref[...])
    mf_ref[...] = jnp.dot(x * ds, wf_ref[...], preferred_element_type=jnp.float32)
    mb_ref[...] = jnp.dot(x * dd, wb_ref[...], preferred_element_type=jnp.float32)


def _mid_body(accf_ref, accb_ref, deg_ref, bf_ref, bb_ref, wf_ref, wb_ref,
              mf_ref, mb_ref):
    ds, dd = _scales(deg_ref[...])
    h = (ALPHA * (accf_ref[...] * dd + bf_ref[...])
         + (1.0 - ALPHA) * (accb_ref[...] * ds + bb_ref[...]))
    h = jnp.maximum(h, 0.0)
    mf_ref[...] = jnp.dot(h * ds, wf_ref[...], preferred_element_type=jnp.float32)
    mb_ref[...] = jnp.dot(h * dd, wb_ref[...], preferred_element_type=jnp.float32)


def _fin_body(accf_ref, accb_ref, deg_ref, bf_ref, bb_ref, out_ref):
    ds, dd = _scales(deg_ref[...])
    out_ref[...] = (ALPHA * (accf_ref[...] * dd + bf_ref[...])
                    + (1.0 - ALPHA) * (accb_ref[...] * ds + bb_ref[...]))


_f32 = jnp.float32
_nd = jax.ShapeDtypeStruct((N, D), _f32)


def kernel(x, edge_index, Wf0, bf0, Wb0, bb0, Wf1, bf1, Wb1, bb1):
    onesD = jnp.ones((K, D), _f32)
    zerosD = jnp.zeros((128, D), _f32)

    src = edge_index[0]
    dst = edge_index[1]
    pad0 = jnp.zeros((EPAD,), jnp.int32)
    padN = jnp.full((EPAD,), N, jnp.int32)
    g0 = jnp.concatenate([src, pad0]).reshape(NCHP, K)
    s0 = jnp.concatenate([dst, padN]).reshape(NCHP, K)
    g1 = jnp.concatenate([dst, pad0]).reshape(NCHP, K)
    s1 = jnp.concatenate([src, padN]).reshape(NCHP, K)
    g0 = g0.reshape(NT, CPT, K)
    s0 = s0.reshape(NT, CPT, K)
    g1 = g1.reshape(NT, CPT, K)
    s1 = s1.reshape(NT, CPT, K)
    p0 = jnp.stack([g0, s0], axis=2)
    p1 = jnp.stack([g1, s1], axis=2)
    eic = jnp.stack([p0, p1]).reshape(2, NT, NB, 6, K)
    es = jnp.stack([s0, s1])

    deg3 = _deg_kernel(es, onesD, zerosD)
    deg2 = jnp.stack([deg3[0, :, 0], deg3[1, :, 0]], axis=1)

    mf0, mb0 = pl.pallas_call(_mm0_body, out_shape=(_nd, _nd))(
        x, deg2, Wf0, Wb0)
    accf0, accb0 = _conv_kernel(mf0, mb0, eic, zerosD)

    mf1, mb1 = pl.pallas_call(_mid_body, out_shape=(_nd, _nd))(
        accf0, accb0, deg2, bf0.reshape(1, D), bb0.reshape(1, D), Wf1, Wb1)
    accf1, accb1 = _conv_kernel(mf1, mb1, eic, zerosD)

    out = pl.pallas_call(_fin_body, out_shape=_nd)(
        accf1, accb1, deg2, bf1.reshape(1, D), bb1.reshape(1, D))
    return out


# speedup vs baseline: 1.9975x; 1.0905x over previous
import functools

import jax
import jax.numpy as jnp
from jax import lax
from jax.experimental import pallas as pl
from jax.experimental.pallas import tpu as pltpu
from jax.experimental.pallas import tpu_sc as plsc

N = 10000
E = 320000
D = 128
ALPHA = 0.5

K = 120
NT = 16
CPT = 168
NB = CPT // 3
NCHP = NT * CPT
EPAD = NCHP * K - E
ACCROWS = 10016
NROWCH = N // 128
NTAIL = N - NROWCH * 128
ZTAIL = ACCROWS - NROWCH * 128

_mesh = plsc.VectorSubcoreMesh(core_axis_name="c", subcore_axis_name="s")


def _zero_acc(zeros_hbm, acc_sp, sid):
    for k in range(5):
        c = sid + k * NT

        @pl.when(c < NROWCH)
        def _():
            pltpu.sync_copy(zeros_hbm, acc_sp.at[pl.ds(c * 128, 128)])

    @pl.when(sid == 1)
    def _():
        pltpu.sync_copy(zeros_hbm.at[pl.ds(0, ZTAIL)],
                        acc_sp.at[pl.ds(NROWCH * 128, ZTAIL)])


def _write_out(acc_sp, out_hbm, sid):
    for k in range(5):
        c = sid + k * NT

        @pl.when(c < NROWCH)
        def _():
            pltpu.sync_copy(acc_sp.at[pl.ds(c * 128, 128)],
                            out_hbm.at[pl.ds(c * 128, 128)])

    @pl.when(sid == 0)
    def _():
        pltpu.sync_copy(acc_sp.at[pl.ds(NROWCH * 128, NTAIL)],
                        out_hbm.at[pl.ds(NROWCH * 128, NTAIL)])


@functools.partial(
    pl.kernel,
    out_type=jax.ShapeDtypeStruct((2, N, D), jnp.float32),
    mesh=_mesh,
    scratch_types=[
        pltpu.VMEM((CPT, K), jnp.int32),
        pltpu.VMEM((K, D), jnp.float32),
        pltpu.VMEM_SHARED((ACCROWS, D), jnp.float32),
        pltpu.SemaphoreType.DMA,
    ],
)
def _deg_kernel(es_hbm, ones_hbm, zeros_hbm, deg_out, idx2, ones_v, acc_sp,
                sem):
    cid = lax.axis_index("c")
    sid = lax.axis_index("s")
    pltpu.sync_copy(ones_hbm, ones_v)
    pltpu.sync_copy(es_hbm.at[1 - cid, sid], idx2)
    _zero_acc(zeros_hbm, acc_sp, sid)
    plsc.subcore_barrier()

    def body(j, carry):
        c = 6 * j
        for b in range(6):
            pltpu.async_copy(ones_v, acc_sp.at[idx2.at[c + b]], sem, add=True)
        for b in range(6):
            pltpu.make_async_copy(ones_v, acc_sp.at[idx2.at[c + b]], sem).wait()
        return carry

    lax.fori_loop(0, CPT // 6, body, 0)
    plsc.subcore_barrier()
    _write_out(acc_sp, deg_out.at[cid], sid)


@functools.partial(
    pl.kernel,
    out_type=(jax.ShapeDtypeStruct((N, D), jnp.float32),
              jax.ShapeDtypeStruct((N, D), jnp.float32)),
    mesh=_mesh,
    scratch_types=[
        pltpu.VMEM((2, 6, K), jnp.int32),
        pltpu.VMEM((3, K, D), jnp.float32),
        pltpu.VMEM_SHARED((ACCROWS, D), jnp.float32),
        pltpu.SemaphoreType.DMA,
        pltpu.SemaphoreType.DMA,
        pltpu.SemaphoreType.DMA,
    ],
)
def _conv_kernel(mf_hbm, mb_hbm, eic_hbm, zeros_hbm, accf_out, accb_out,
                 ibuf, gbuf, acc_sp, gsem, ssem, isem):
    cid = lax.axis_index("c")
    sid = lax.axis_index("s")

    def run(m_hbm, out_hbm):
        _zero_acc(zeros_hbm, acc_sp, sid)
        plsc.subcore_barrier()

        pltpu.sync_copy(eic_hbm.at[cid, sid, 0], ibuf.at[0])

        def batch(a, nb_idx):
            g = [pltpu.async_copy(m_hbm.at[ibuf.at[a, 2 * b]], gbuf.at[b],
                                  gsem) for b in range(3)]
            ipf = pltpu.async_copy(eic_hbm.at[cid, sid, nb_idx],
                                   ibuf.at[1 - a], isem)
            s = []
            for b in range(3):
                g[b].wait()
                s.append(pltpu.async_copy(gbuf.at[b],
                                          acc_sp.at[ibuf.at[a, 2 * b + 1]],
                                          ssem, add=True))
            for d in s:
                d.wait()
            ipf.wait()

        def body(j, carry):
            b0 = 2 * j
            batch(0, b0 + 1)
            batch(1, jnp.minimum(b0 + 2, NB - 1))
            return carry

        lax.fori_loop(0, NB // 2, body, 0)
        plsc.subcore_barrier()
        _write_out(acc_sp, out_hbm, sid)

    @pl.when(cid == 0)
    def _():
        run(mf_hbm, accf_out)

    @pl.when(cid == 1)
    def _():
        run(mb_hbm, accb_out)


def _scales(deg2):
    ds = lax.rsqrt(jnp.maximum(deg2[:, 0:1], 1.0))
    dd = lax.rsqrt(jnp.maximum(deg2[:, 1:2], 1.0))
    return ds, dd


def _mm0_body(x_ref, deg_ref, wf_ref, wb_ref, mf_ref, mb_ref):
    x = x_ref[...]
    ds, dd = _scales(deg_ref[...])
    mf_ref[...] = jnp.dot(x * ds, wf_ref[...], preferred_element_type=jnp.float32)
    mb_ref[...] = jnp.dot(x * dd, wb_ref[...], preferred_element_type=jnp.float32)


def _mid_body(accf_ref, accb_ref, deg_ref, bf_ref, bb_ref, wf_ref, wb_ref,
              mf_ref, mb_ref):
    ds, dd = _scales(deg_ref[...])
    h = (ALPHA * (accf_ref[...] * dd + bf_ref[...])
         + (1.0 - ALPHA) * (accb_ref[...] * ds + bb_ref[...]))
    h = jnp.maximum(h, 0.0)
    mf_ref[...] = jnp.dot(h * ds, wf_ref[...], preferred_element_type=jnp.float32)
    mb_ref[...] = jnp.dot(h * dd, wb_ref[...], preferred_element_type=jnp.float32)


def _fin_body(accf_ref, accb_ref, deg_ref, bf_ref, bb_ref, out_ref):
    ds, dd = _scales(deg_ref[...])
    out_ref[...] = (ALPHA * (accf_ref[...] * dd + bf_ref[...])
                    + (1.0 - ALPHA) * (accb_ref[...] * ds + bb_ref[...]))


_f32 = jnp.float32
_nd = jax.ShapeDtypeStruct((N, D), _f32)


def kernel(x, edge_index, Wf0, bf0, Wb0, bb0, Wf1, bf1, Wb1, bb1):
    onesD = jnp.ones((K, D), _f32)
    zerosD = jnp.zeros((128, D), _f32)

    src = edge_index[0]
    dst = edge_index[1]
    pad0 = jnp.zeros((EPAD,), jnp.int32)
    padN = jnp.full((EPAD,), N, jnp.int32)
    g0 = jnp.concatenate([src, pad0]).reshape(NCHP, K)
    s0 = jnp.concatenate([dst, padN]).reshape(NCHP, K)
    g1 = jnp.concatenate([dst, pad0]).reshape(NCHP, K)
    s1 = jnp.concatenate([src, padN]).reshape(NCHP, K)
    g0 = g0.reshape(NT, CPT, K)
    s0 = s0.reshape(NT, CPT, K)
    g1 = g1.reshape(NT, CPT, K)
    s1 = s1.reshape(NT, CPT, K)
    p0 = jnp.stack([g0, s0], axis=2)
    p1 = jnp.stack([g1, s1], axis=2)
    eic = jnp.stack([p0, p1]).reshape(2, NT, NB, 6, K)
    es = jnp.stack([s0, s1])

    deg3 = _deg_kernel(es, onesD, zerosD)
    deg2 = jnp.stack([deg3[0, :, 0], deg3[1, :, 0]], axis=1)

    mf0, mb0 = pl.pallas_call(_mm0_body, out_shape=(_nd, _nd))(
        x, deg2, Wf0, Wb0)
    accf0, accb0 = _conv_kernel(mf0, mb0, eic, zerosD)

    mf1, mb1 = pl.pallas_call(_mid_body, out_shape=(_nd, _nd))(
        accf0, accb0, deg2, bf0.reshape(1, D), bb0.reshape(1, D), Wf1, Wb1)
    accf1, accb1 = _conv_kernel(mf1, mb1, eic, zerosD)

    out = pl.pallas_call(_fin_body, out_shape=_nd)(
        accf1, accb1, deg2, bf1.reshape(1, D), bb1.reshape(1, D))
    return out
